# baseline (speedup 1.0000x reference)
import jax
import jax.numpy as jnp
from jax.experimental import pallas as pl
from jax.experimental.pallas import tpu as pltpu

_BB = 8
_PB = 2944


def _make_body(num_p):
    def _body(x_ref, y_ref, l_ref, f_ref, out_ref, acc_m, acc_c, acc_l):
        bi = pl.program_id(0)
        pj = pl.program_id(1)
        nbi = pl.num_programs(0)
        npj = pl.num_programs(1)

        nc = x_ref.shape[0]
        l = l_ref[...]
        f = f_ref[...]

        lane = jax.lax.broadcasted_iota(jnp.int32, (_BB, _PB), 1)
        valid = (pj * _PB + lane) < num_p

        bg = x_ref[0]
        fg = x_ref[1]
        tsum = None
        for s in range(nc):
            xs = x_ref[s]
            ys = y_ref[s]
            ts = (xs - ys) * (jnp.log(xs + 1e-7) - jnp.log(ys + 1e-7))
            tsum = ts if tsum is None else tsum + ts
            if s >= 2:
                fg = jnp.maximum(fg, xs)
        mb = (fg > bg) & valid

        d = l - f
        rloc = jnp.sum(d * d, axis=1) + 4.0 * l[:, 0] * f[:, 0]

        m_c = jnp.where(mb, 1.0, 0.0)
        c_c = jnp.where(mb, tsum, 0.0)
        l_c = jnp.where(mb, rloc, 0.0)

        first = (bi == 0) & (pj == 0)

        @pl.when(first)
        def _():
            acc_m[...] = m_c
            acc_c[...] = c_c
            acc_l[...] = l_c

        @pl.when(jnp.logical_not(first))
        def _():
            acc_m[...] = acc_m[...] + m_c
            acc_c[...] = acc_c[...] + c_c
            acc_l[...] = acc_l[...] + l_c

        @pl.when((bi == nbi - 1) & (pj == npj - 1))
        def _():
            out_ref[...] = jnp.stack([
                jnp.sum(acc_m[...]),
                jnp.sum(acc_c[...]),
                jnp.sum(acc_l[...]),
            ]).reshape(1, 3)

    return _body


def kernel(conf, conf_flip, loc, loc_flip):
    b, num_p, c = conf.shape

    xt = conf.transpose(2, 0, 1)
    yt = conf_flip.transpose(2, 0, 1)
    lt = loc.transpose(0, 2, 1)
    ft = loc_flip.transpose(0, 2, 1)

    grid = (b // _BB, -(-num_p // _PB))
    out = pl.pallas_call(
        _make_body(num_p),
        grid=grid,
        in_specs=[
            pl.BlockSpec((c, _BB, _PB), lambda i, j: (0, i, j)),
            pl.BlockSpec((c, _BB, _PB), lambda i, j: (0, i, j)),
            pl.BlockSpec((_BB, 4, _PB), lambda i, j: (i, 0, j)),
            pl.BlockSpec((_BB, 4, _PB), lambda i, j: (i, 0, j)),
        ],
        out_specs=pl.BlockSpec((1, 3), lambda i, j: (0, 0)),
        out_shape=jax.ShapeDtypeStruct((1, 3), jnp.float32),
        scratch_shapes=[
            pltpu.VMEM((_BB, _PB), jnp.float32),
            pltpu.VMEM((_BB, _PB), jnp.float32),
            pltpu.VMEM((_BB, _PB), jnp.float32),
        ],
        compiler_params=pltpu.CompilerParams(
            dimension_semantics=("arbitrary", "arbitrary"),
        ),
    )(xt, yt, lt, ft)

    total = jnp.maximum(out[0, 0], 1.0)
    return out[0, 1] / (2.0 * total) + out[0, 2] / (4.0 * total)


# speedup vs baseline: 1.0669x; 1.0669x over previous
import jax
import jax.numpy as jnp
from jax.experimental import pallas as pl
from jax.experimental.pallas import tpu as pltpu

_BB = 16
_PB = 2944


def _make_body(num_p):
    def _body(x_ref, y_ref, l_ref, f_ref, out_ref, acc_m, acc_c, acc_l):
        bi = pl.program_id(0)
        pj = pl.program_id(1)
        nbi = pl.num_programs(0)
        npj = pl.num_programs(1)

        nc = x_ref.shape[0]
        l = l_ref[...]
        f = f_ref[...]

        lane = jax.lax.broadcasted_iota(jnp.int32, (_BB, _PB), 1)
        valid = (pj * _PB + lane) < num_p

        bg = x_ref[0]
        fg = x_ref[1]
        tsum = None
        for s in range(nc):
            xs = x_ref[s]
            ys = y_ref[s]
            ts = (xs - ys) * (jnp.log(xs + 1e-7) - jnp.log(ys + 1e-7))
            tsum = ts if tsum is None else tsum + ts
            if s >= 2:
                fg = jnp.maximum(fg, xs)
        mb = (fg > bg) & valid

        d = l - f
        rloc = jnp.sum(d * d, axis=1) + 4.0 * l[:, 0] * f[:, 0]

        m_c = jnp.where(mb, 1.0, 0.0)
        c_c = jnp.where(mb, tsum, 0.0)
        l_c = jnp.where(mb, rloc, 0.0)

        first = (bi == 0) & (pj == 0)

        @pl.when(first)
        def _():
            acc_m[...] = m_c
            acc_c[...] = c_c
            acc_l[...] = l_c

        @pl.when(jnp.logical_not(first))
        def _():
            acc_m[...] = acc_m[...] + m_c
            acc_c[...] = acc_c[...] + c_c
            acc_l[...] = acc_l[...] + l_c

        @pl.when((bi == nbi - 1) & (pj == npj - 1))
        def _():
            out_ref[...] = jnp.stack([
                jnp.sum(acc_m[...]),
                jnp.sum(acc_c[...]),
                jnp.sum(acc_l[...]),
            ]).reshape(1, 3)

    return _body


def kernel(conf, conf_flip, loc, loc_flip):
    b, num_p, c = conf.shape

    xt = conf.transpose(2, 0, 1)
    yt = conf_flip.transpose(2, 0, 1)
    lt = loc.transpose(0, 2, 1)
    ft = loc_flip.transpose(0, 2, 1)

    grid = (b // _BB, -(-num_p // _PB))
    out = pl.pallas_call(
        _make_body(num_p),
        grid=grid,
        in_specs=[
            pl.BlockSpec((c, _BB, _PB), lambda i, j: (0, i, j)),
            pl.BlockSpec((c, _BB, _PB), lambda i, j: (0, i, j)),
            pl.BlockSpec((_BB, 4, _PB), lambda i, j: (i, 0, j)),
            pl.BlockSpec((_BB, 4, _PB), lambda i, j: (i, 0, j)),
        ],
        out_specs=pl.BlockSpec((1, 3), lambda i, j: (0, 0)),
        out_shape=jax.ShapeDtypeStruct((1, 3), jnp.float32),
        scratch_shapes=[
            pltpu.VMEM((_BB, _PB), jnp.float32),
            pltpu.VMEM((_BB, _PB), jnp.float32),
            pltpu.VMEM((_BB, _PB), jnp.float32),
        ],
        compiler_params=pltpu.CompilerParams(
            dimension_semantics=("arbitrary", "arbitrary"),
        ),
    )(xt, yt, lt, ft)

    total = jnp.maximum(out[0, 0], 1.0)
    return out[0, 1] / (2.0 * total) + out[0, 2] / (4.0 * total)
